# baseline (speedup 1.0000x reference)
import functools

import jax
import jax.numpy as jnp
from jax import lax
from jax.experimental import pallas as pl
from jax.experimental.pallas import tpu as pltpu
from jax.experimental.pallas import tpu_sc as plsc

ROWS = 64
N = 32768
L = 16
NSLICES = N // L
WORKERS = 32
ROWS_PER_WORKER = ROWS // WORKERS
NEG = -3.0e38
U = 8
CHUNKS = 4
CHUNK = N // CHUNKS
CGROUPS = CHUNK // (U * L)


def _compact_chunk(buf, cand_v, base_g, carry):
    def cpt_body(i, carry):
        off, w = carry
        vs = [buf[pl.ds(((base_g + i) * U + u) * L, L)] for u in range(U)]
        ps = [v > w for v in vs]
        for u in range(U):
            off_s = off[0]
            plsc.store_compressed(
                cand_v.at[pl.ds(off_s, L)], vs[u], mask=ps[u]
            )
            off = off + plsc.all_reduce_population_count(ps[u])
        wa = jnp.maximum(jnp.maximum(vs[0], vs[1]),
                         jnp.maximum(vs[2], vs[3]))
        wb = jnp.maximum(jnp.maximum(vs[4], vs[5]),
                         jnp.maximum(vs[6], vs[7]))
        w = jnp.maximum(w, jnp.maximum(wa, wb) - 1.0)
        return (off, w)

    return lax.fori_loop(base_g, base_g + CGROUPS, cpt_body, carry, unroll=1)


def _issue_in(x_hbm, row, buf, sems_in):
    return [
        pltpu.async_copy(
            x_hbm.at[row, pl.ds(c * CHUNK, CHUNK)],
            buf.at[pl.ds(c * CHUNK, CHUNK)],
            sems_in[c],
        )
        for c in range(CHUNKS)
    ]


def _process_row(out_hbm, row, buf, cand_v, in_copies, sems_out):
    carry = (jnp.zeros((L,), jnp.int32), jnp.full((L,), NEG, jnp.float32))
    for c in range(CHUNKS):
        in_copies[c].wait()
        carry = _compact_chunk(buf, cand_v, c * CGROUPS, carry)
    off16, _ = carry
    k_count = off16[0]
    cand_v[pl.ds(k_count, L)] = jnp.full((L,), NEG, jnp.float32)
    nsl = (k_count + (L - 1)) >> 4

    def max_body(i, acc):
        return jnp.maximum(acc, cand_v[pl.ds(i * L, L)])

    m16 = lax.fori_loop(0, nsl, max_body, jnp.full((L,), NEG, jnp.float32))
    m = lax.broadcast_in_dim(jnp.max(m16), (L,), ())

    def n_cond(carry):
        t, t_prev = carry
        return jnp.all(t > t_prev)

    def n_body(carry):
        t, _ = carry

        def pass_body(i, acc):
            sa, ca = acc
            v = cand_v[pl.ds(i * L, L)]
            d = v - t
            sa = sa + jnp.maximum(d, 0.0)
            ca = ca + plsc.all_reduce_population_count(d > 0.0)
            return (sa, ca)

        sa, ca = lax.fori_loop(
            0,
            nsl,
            pass_body,
            (jnp.zeros((L,), jnp.float32), jnp.zeros((L,), jnp.int32)),
        )
        s = lax.broadcast_in_dim(jnp.sum(sa), (L,), ())
        c = ca.astype(jnp.float32)
        t_new = t + (s - 1.0) / c
        return (jnp.where(t_new > t, t_new, t), t)

    tau, _ = lax.while_loop(
        n_cond, n_body, (m - 1.0, jnp.full((L,), NEG, jnp.float32))
    )

    out_copies = []
    for c in range(CHUNKS):

        def out_body(i, carry):
            for u in range(U):
                sl = pl.ds((i * U + u) * L, L)
                buf[sl] = jnp.maximum(buf[sl] - tau, 0.0)
            return carry

        lax.fori_loop(c * CGROUPS, (c + 1) * CGROUPS, out_body, 0, unroll=1)
        out_copies.append(
            pltpu.async_copy(
                buf.at[pl.ds(c * CHUNK, CHUNK)],
                out_hbm.at[row, pl.ds(c * CHUNK, CHUNK)],
                sems_out[c],
            )
        )
    return out_copies


def _sparsemax_body(x_hbm, out_hbm, buf_a, buf_b, cand_v, *sems):
    wid = lax.axis_index("s") * 2 + lax.axis_index("c")
    row0 = wid * ROWS_PER_WORKER
    row1 = row0 + 1
    in_a = _issue_in(x_hbm, row0, buf_a, sems[0:4])
    in_b = _issue_in(x_hbm, row1, buf_b, sems[4:8])
    outs_a = _process_row(out_hbm, row0, buf_a, cand_v, in_a, sems[8:12])
    outs_b = _process_row(out_hbm, row1, buf_b, cand_v, in_b, sems[12:16])
    for cp in outs_a + outs_b:
        cp.wait()


@jax.jit
def kernel(x):
    return pl.kernel(
        _sparsemax_body,
        out_type=jax.ShapeDtypeStruct((ROWS, N), jnp.float32),
        mesh=plsc.VectorSubcoreMesh(core_axis_name="c", subcore_axis_name="s"),
        scratch_types=[
            pltpu.VMEM((N,), jnp.float32),
            pltpu.VMEM((N,), jnp.float32),
            pltpu.VMEM((N + L,), jnp.float32),
        ]
        + [pltpu.SemaphoreType.DMA] * 16,
        compiler_params=pltpu.CompilerParams(needs_layout_passes=False),
    )(x)
